# baseline (speedup 1.0000x reference)
import functools

import jax
import jax.numpy as jnp
from jax import lax
from jax.experimental import pallas as pl
from jax.experimental.pallas import tpu as pltpu
from jax.experimental.pallas import tpu_sc as plsc

N = 10000
E = 320000
D_IN = 128
H = 32
G = 64
C = 2

NC = 2
NS = 16
NW = NC * NS
CH = 128
GPT = 80
EPT = GPT * CH
EP = NW * EPT
NPAD = 10240

_mesh = plsc.VectorSubcoreMesh(core_axis_name="c", subcore_axis_name="s")
_sc_params = pltpu.CompilerParams(use_tc_tiling_on_sc=False,
                                  needs_layout_passes=False)

F32 = jnp.float32


def _mm(a, w):
    return jnp.dot(a.astype(jnp.bfloat16), w.astype(jnp.bfloat16),
                   preferred_element_type=F32)


@functools.partial(
    pl.kernel,
    mesh=_mesh,
    out_type=jax.ShapeDtypeStruct((NC, N, H), F32),
    compiler_params=_sc_params,
    scratch_types=[
        pltpu.VMEM_SHARED((N, H), F32),
        pltpu.VMEM((GPT, CH), jnp.int32),
        pltpu.VMEM((GPT, CH), jnp.int32),
        pltpu.VMEM((EPT,), F32),
        pltpu.VMEM((4, CH, H), F32),
        pltpu.VMEM((4, CH, H), F32),
        pltpu.SemaphoreType.DMA,
        pltpu.SemaphoreType.DMA,
        pltpu.SemaphoreType.DMA,
        pltpu.SemaphoreType.DMA,
        pltpu.SemaphoreType.DMA,
        pltpu.SemaphoreType.DMA,
        pltpu.SemaphoreType.DMA,
        pltpu.SemaphoreType.DMA,
    ],
)
def _sc_agg(y_hbm, row_hbm, col_hbm, ew_hbm, z_hbm, out_hbm,
            acc, rowbuf, colbuf, ewbuf, gbuf, sbuf,
            gsem0, gsem1, gsem2, gsem3, ssem0, ssem1, ssem2, ssem3):
    cid = lax.axis_index("c")
    sid = lax.axis_index("s")
    wid = cid * NS + sid
    spt = N // NS
    sl = pl.ds(sid * spt, spt)
    gsems = (gsem0, gsem1, gsem2, gsem3)
    ssems = (ssem0, ssem1, ssem2, ssem3)

    pltpu.sync_copy(z_hbm.at[sl, :], acc.at[sl, :])
    pltpu.sync_copy(row_hbm.at[wid], rowbuf)
    pltpu.sync_copy(col_hbm.at[wid], colbuf)
    pltpu.sync_copy(ew_hbm.at[wid], ewbuf)
    plsc.subcore_barrier()

    def start_gather(g, k):
        pltpu.async_copy(y_hbm.at[rowbuf.at[g]], gbuf.at[k], gsems[k])

    def wait_gather(k):
        pltpu.make_async_copy(y_hbm.at[rowbuf.at[0]], gbuf.at[k],
                              gsems[k]).wait()

    def start_scatter(g, k):
        pltpu.async_copy(sbuf.at[k], acc.at[colbuf.at[g]], ssems[k],
                         add=True)

    def wait_scatter(k):
        pltpu.make_async_copy(sbuf.at[k], acc.at[colbuf.at[0]],
                              ssems[k]).wait()

    for k0 in range(4):
        start_gather(k0, k0)

    @pl.loop(0, GPT, step=4)
    def _(g):
        for k in (0, 1, 2, 3):
            cur = g + k
            wait_gather(k)

            @pl.when(g > 0)
            def _():
                wait_scatter(k)

            src = gbuf.at[k]
            dst = sbuf.at[k]

            @pl.loop(0, CH, step=8)
            def _(eb):
                base = cur * CH + eb
                for j in range(8):
                    iv = lax.broadcast_in_dim(base + j, (16,), ())
                    w16 = plsc.load_gather(ewbuf, [iv])
                    e = eb + j
                    dst[e, pl.ds(0, 16)] = src[e, pl.ds(0, 16)] * w16
                    dst[e, pl.ds(16, 16)] = src[e, pl.ds(16, 16)] * w16

            @pl.when(cur + 4 < GPT)
            def _():
                start_gather(cur + 4, k)

            start_scatter(cur, k)

    for k0 in range(4):
        wait_scatter(k0)
    plsc.subcore_barrier()
    pltpu.sync_copy(acc.at[sl, :], out_hbm.at[cid, sl, :])


@functools.partial(
    pl.kernel,
    mesh=_mesh,
    out_type=jax.ShapeDtypeStruct((NC, N, H), F32),
    compiler_params=_sc_params,
    scratch_types=[
        pltpu.VMEM_SHARED((N, H), F32),
        pltpu.VMEM((GPT, CH), jnp.int32),
        pltpu.VMEM((4, CH, H), F32),
        pltpu.SemaphoreType.DMA,
        pltpu.SemaphoreType.DMA,
        pltpu.SemaphoreType.DMA,
        pltpu.SemaphoreType.DMA,
        pltpu.SemaphoreType.DMA,
        pltpu.SemaphoreType.DMA,
        pltpu.SemaphoreType.DMA,
        pltpu.SemaphoreType.DMA,
    ],
)
def _sc_deg(ewx_hbm, col_hbm, z_hbm, out_hbm,
            acc, colbuf, wbuf,
            gsem0, gsem1, gsem2, gsem3, ssem0, ssem1, ssem2, ssem3):
    cid = lax.axis_index("c")
    sid = lax.axis_index("s")
    wid = cid * NS + sid
    spt = N // NS
    sl = pl.ds(sid * spt, spt)
    gsems = (gsem0, gsem1, gsem2, gsem3)
    ssems = (ssem0, ssem1, ssem2, ssem3)

    pltpu.sync_copy(z_hbm.at[sl, :], acc.at[sl, :])
    pltpu.sync_copy(col_hbm.at[wid], colbuf)
    plsc.subcore_barrier()

    def start_load(g, k):
        pltpu.async_copy(ewx_hbm.at[wid, g], wbuf.at[k], gsems[k])

    def wait_load(k):
        pltpu.make_async_copy(ewx_hbm.at[wid, 0], wbuf.at[k],
                              gsems[k]).wait()

    def start_scatter(g, k):
        pltpu.async_copy(wbuf.at[k], acc.at[colbuf.at[g]], ssems[k],
                         add=True)

    def wait_scatter(k):
        pltpu.make_async_copy(wbuf.at[k], acc.at[colbuf.at[0]],
                              ssems[k]).wait()

    start_load(0, 0)
    start_load(1, 1)

    @pl.loop(0, GPT, step=4)
    def _(g):
        for j in (0, 1, 2, 3):
            cur = g + j
            wait_load(j)
            start_scatter(cur, j)
            nj = (j + 2) % 4

            @pl.when(cur + 2 < GPT)
            def _():
                @pl.when(cur >= 2)
                def _():
                    wait_scatter(nj)

                start_load(cur + 2, nj)

    for k0 in range(4):
        wait_scatter(k0)
    plsc.subcore_barrier()
    pltpu.sync_copy(acc.at[sl, :], out_hbm.at[cid, sl, :])


def _tc_first_body(x_ref, w_ref, dp_ref, y_ref, dinv_ref):
    d = 1.0 + dp_ref[0, :, 0:1] + dp_ref[1, :, 0:1]
    dinv = lax.rsqrt(d)
    xw = _mm(x_ref[...], w_ref[...])
    y_ref[...] = dinv * xw
    dinv_ref[...] = dinv


def _tc_mid_body(p_ref, y_ref, dinv_ref, cb_ref, g_ref, bb_ref, w_ref,
                 yo_ref):
    dinv = dinv_ref[...]
    t = dinv * (p_ref[0] + p_ref[1] + y_ref[...]) + cb_ref[...]
    m = jnp.mean(t, axis=0, keepdims=True)
    tc = t - m
    v = jnp.mean(tc * tc, axis=0, keepdims=True)
    h = jnp.maximum(g_ref[...] * tc * lax.rsqrt(v + 1e-5) + bb_ref[...],
                    0.0)
    yo_ref[...] = dinv * _mm(h, w_ref[...])


def _tc_final_body(p_ref, y_ref, dinv_ref, cb_ref, b_ref,
                   w0_ref, b0_ref, g_ref, bb_ref, w1_ref, b1_ref, out_ref):
    t = dinv_ref[...] * (p_ref[0] + p_ref[1] + y_ref[...]) + cb_ref[...]
    oh = (lax.broadcasted_iota(jnp.int32, (G, N), 0)
          == b_ref[...]).astype(F32)
    pooled = jnp.dot(oh, t, preferred_element_type=F32, precision=lax.Precision.HIGHEST)
    z = _mm(pooled, w0_ref[...]) + b0_ref[...]
    m = jnp.mean(z, axis=0, keepdims=True)
    zc = z - m
    v = jnp.mean(zc * zc, axis=0, keepdims=True)
    z = jnp.maximum(g_ref[...] * zc * lax.rsqrt(v + 1e-5) + bb_ref[...],
                    0.0)
    out_ref[...] = _mm(z, w1_ref[...]) + b1_ref[...]


_tc_first = pl.pallas_call(
    _tc_first_body,
    out_shape=[jax.ShapeDtypeStruct((N, H), F32),
               jax.ShapeDtypeStruct((N, 1), F32)],
)

_tc_mid = pl.pallas_call(
    _tc_mid_body,
    out_shape=jax.ShapeDtypeStruct((N, H), F32),
)

_tc_final = pl.pallas_call(
    _tc_final_body,
    out_shape=jax.ShapeDtypeStruct((G, C), F32),
)


def kernel(x, edge_index, edge_attr, batch, params):
    row = edge_index[0]
    col = edge_index[1]
    pad = EP - E
    pidx = jnp.arange(pad, dtype=jnp.int32) % N
    rowp = jnp.concatenate([row, pidx]).reshape(NW, GPT, CH)
    colp = jnp.concatenate([col, pidx]).reshape(NW, GPT, CH)
    ewp = jnp.concatenate([edge_attr.astype(F32), jnp.zeros((pad,), F32)])
    ew_flat = ewp.reshape(NW, EPT)
    z_acc = jnp.zeros((N, H), F32)
    batch2d = batch.reshape(1, N)

    def cvec(v):
        return v.reshape(1, -1).astype(F32)

    ewx = jnp.broadcast_to(ewp[:, None], (EP, H)).reshape(NW, GPT, CH, H)
    dp = _sc_deg(ewx, colp, z_acc)
    y, dinv = _tc_first(x, params['conv0_W'], dp)
    for i in range(3):
        p = _sc_agg(y, rowp, colp, ew_flat, z_acc)
        y = _tc_mid(p, y, dinv, cvec(params[f'conv{i}_b']),
                    cvec(params[f'bn{i}_g']), cvec(params[f'bn{i}_b']),
                    params[f'conv{i + 1}_W'])
    p = _sc_agg(y, rowp, colp, ew_flat, z_acc)
    out = _tc_final(p, y, dinv, cvec(params['conv3_b']), batch2d,
                    params['mlp0_W'], cvec(params['mlp0_b']),
                    cvec(params['mlp_bn_g']), cvec(params['mlp_bn_b']),
                    params['mlp1_W'], cvec(params['mlp1_b']))
    return out


# speedup vs baseline: 1.2563x; 1.2563x over previous
import functools

import jax
import jax.numpy as jnp
from jax import lax
from jax.experimental import pallas as pl
from jax.experimental.pallas import tpu as pltpu
from jax.experimental.pallas import tpu_sc as plsc

N = 10000
E = 320000
D_IN = 128
H = 32
G = 64
C = 2

NC = 2
NS = 16
NW = NC * NS
CH = 128
GPT = 80
EPT = GPT * CH
EP = NW * EPT
NPAD = 10240

_mesh = plsc.VectorSubcoreMesh(core_axis_name="c", subcore_axis_name="s")
_sc_params = pltpu.CompilerParams(use_tc_tiling_on_sc=False,
                                  needs_layout_passes=False)

F32 = jnp.float32


def _mm(a, w):
    return jnp.dot(a.astype(jnp.bfloat16), w.astype(jnp.bfloat16),
                   preferred_element_type=F32)


@functools.partial(
    pl.kernel,
    mesh=_mesh,
    out_type=jax.ShapeDtypeStruct((NC, N, H), F32),
    compiler_params=_sc_params,
    scratch_types=[
        pltpu.VMEM_SHARED((N, H), F32),
        pltpu.VMEM((GPT, CH), jnp.int32),
        pltpu.VMEM((GPT, CH), jnp.int32),
        pltpu.VMEM((EPT,), F32),
        pltpu.VMEM((4, CH, H), F32),
        pltpu.VMEM((4, CH, H), F32),
        pltpu.SemaphoreType.DMA,
        pltpu.SemaphoreType.DMA,
        pltpu.SemaphoreType.DMA,
        pltpu.SemaphoreType.DMA,
        pltpu.SemaphoreType.DMA,
        pltpu.SemaphoreType.DMA,
        pltpu.SemaphoreType.DMA,
        pltpu.SemaphoreType.DMA,
    ],
)
def _sc_agg(y_hbm, row_hbm, col_hbm, ew_hbm, z_hbm, out_hbm,
            acc, rowbuf, colbuf, ewbuf, gbuf, sbuf,
            gsem0, gsem1, gsem2, gsem3, ssem0, ssem1, ssem2, ssem3):
    cid = lax.axis_index("c")
    sid = lax.axis_index("s")
    wid = cid * NS + sid
    spt = N // NS
    sl = pl.ds(sid * spt, spt)
    gsems = (gsem0, gsem1, gsem2, gsem3)
    ssems = (ssem0, ssem1, ssem2, ssem3)

    pltpu.sync_copy(z_hbm.at[sl, :], acc.at[sl, :])
    pltpu.sync_copy(row_hbm.at[wid], rowbuf)
    pltpu.sync_copy(col_hbm.at[wid], colbuf)
    pltpu.sync_copy(ew_hbm.at[wid], ewbuf)
    plsc.subcore_barrier()

    def start_gather(g, k):
        pltpu.async_copy(y_hbm.at[rowbuf.at[g]], gbuf.at[k], gsems[k])

    def wait_gather(k):
        pltpu.make_async_copy(y_hbm.at[rowbuf.at[0]], gbuf.at[k],
                              gsems[k]).wait()

    def start_scatter(g, k):
        pltpu.async_copy(sbuf.at[k], acc.at[colbuf.at[g]], ssems[k],
                         add=True)

    def wait_scatter(k):
        pltpu.make_async_copy(sbuf.at[k], acc.at[colbuf.at[0]],
                              ssems[k]).wait()

    for k0 in range(4):
        start_gather(k0, k0)

    @pl.loop(0, GPT, step=4)
    def _(g):
        for k in (0, 1, 2, 3):
            cur = g + k
            wait_gather(k)

            @pl.when(g > 0)
            def _():
                wait_scatter(k)

            src = gbuf.at[k]
            dst = sbuf.at[k]

            @pl.loop(0, CH, step=8)
            def _(eb):
                base = cur * CH + eb
                for j in range(8):
                    iv = lax.broadcast_in_dim(base + j, (16,), ())
                    w16 = plsc.load_gather(ewbuf, [iv])
                    e = eb + j
                    dst[e, pl.ds(0, 16)] = src[e, pl.ds(0, 16)] * w16
                    dst[e, pl.ds(16, 16)] = src[e, pl.ds(16, 16)] * w16

            @pl.when(cur + 4 < GPT)
            def _():
                start_gather(cur + 4, k)

            start_scatter(cur, k)

    for k0 in range(4):
        wait_scatter(k0)
    plsc.subcore_barrier()
    pltpu.sync_copy(acc.at[sl, :], out_hbm.at[cid, sl, :])


def _tc_first_body(x_ref, w_ref, dp_ref, y_ref, dinv_ref):
    d = 1.0 + dp_ref[0, :, 0:1] + dp_ref[1, :, 0:1]
    dinv = lax.rsqrt(d)
    xw = _mm(x_ref[...], w_ref[...])
    y_ref[...] = dinv * xw
    dinv_ref[...] = dinv


def _tc_mid_body(p_ref, y_ref, dinv_ref, cb_ref, g_ref, bb_ref, w_ref,
                 yo_ref):
    dinv = dinv_ref[...]
    t = dinv * (p_ref[0] + p_ref[1] + y_ref[...]) + cb_ref[...]
    m = jnp.mean(t, axis=0, keepdims=True)
    tc = t - m
    v = jnp.mean(tc * tc, axis=0, keepdims=True)
    h = jnp.maximum(g_ref[...] * tc * lax.rsqrt(v + 1e-5) + bb_ref[...],
                    0.0)
    yo_ref[...] = dinv * _mm(h, w_ref[...])


def _tc_final_body(p_ref, y_ref, dinv_ref, cb_ref, b_ref,
                   w0_ref, b0_ref, g_ref, bb_ref, w1_ref, b1_ref, out_ref):
    t = dinv_ref[...] * (p_ref[0] + p_ref[1] + y_ref[...]) + cb_ref[...]
    oh = (lax.broadcasted_iota(jnp.int32, (G, N), 0)
          == b_ref[...]).astype(F32)
    pooled = jnp.dot(oh, t, preferred_element_type=F32, precision=lax.Precision.HIGHEST)
    z = _mm(pooled, w0_ref[...]) + b0_ref[...]
    m = jnp.mean(z, axis=0, keepdims=True)
    zc = z - m
    v = jnp.mean(zc * zc, axis=0, keepdims=True)
    z = jnp.maximum(g_ref[...] * zc * lax.rsqrt(v + 1e-5) + bb_ref[...],
                    0.0)
    out_ref[...] = _mm(z, w1_ref[...]) + b1_ref[...]


_tc_first = pl.pallas_call(
    _tc_first_body,
    out_shape=[jax.ShapeDtypeStruct((N, H), F32),
               jax.ShapeDtypeStruct((N, 1), F32)],
)

_tc_mid = pl.pallas_call(
    _tc_mid_body,
    out_shape=jax.ShapeDtypeStruct((N, H), F32),
)

_tc_final = pl.pallas_call(
    _tc_final_body,
    out_shape=jax.ShapeDtypeStruct((G, C), F32),
)


def kernel(x, edge_index, edge_attr, batch, params):
    row = edge_index[0]
    col = edge_index[1]
    pad = EP - E
    pidx = jnp.arange(pad, dtype=jnp.int32) % N
    rowp = jnp.concatenate([row, pidx]).reshape(NW, GPT, CH)
    colp = jnp.concatenate([col, pidx]).reshape(NW, GPT, CH)
    ewp = jnp.concatenate([edge_attr.astype(F32), jnp.zeros((pad,), F32)])
    ew_flat = ewp.reshape(NW, EPT)
    z_acc = jnp.zeros((N, H), F32)
    batch2d = batch.reshape(1, N)

    def cvec(v):
        return v.reshape(1, -1).astype(F32)

    ones_y = jnp.ones((N, H), F32)
    dp = _sc_agg(ones_y, colp, colp, ew_flat, z_acc)
    y, dinv = _tc_first(x, params['conv0_W'], dp)
    for i in range(3):
        p = _sc_agg(y, rowp, colp, ew_flat, z_acc)
        y = _tc_mid(p, y, dinv, cvec(params[f'conv{i}_b']),
                    cvec(params[f'bn{i}_g']), cvec(params[f'bn{i}_b']),
                    params[f'conv{i + 1}_W'])
    p = _sc_agg(y, rowp, colp, ew_flat, z_acc)
    out = _tc_final(p, y, dinv, cvec(params['conv3_b']), batch2d,
                    params['mlp0_W'], cvec(params['mlp0_b']),
                    cvec(params['mlp_bn_g']), cvec(params['mlp_bn_b']),
                    params['mlp1_W'], cvec(params['mlp1_b']))
    return out
